# baseline (speedup 1.0000x reference)
import functools

import jax
import jax.numpy as jnp
from jax import lax
from jax.experimental import pallas as pl
from jax.experimental.pallas import tpu as pltpu
from jax.experimental.pallas import tpu_sc as plsc

N_NODES = 100000
D = 128
NF = D // 16
NSEG = 128
NC, NS = 2, 16
NW = NC * NS
CHUNK = 3128
BLK = 128
NBLK = 26
LAST_START = N_NODES - BLK
NEG_INF = float("-inf")


def _sc_partial_max(x, batch_i32):
    mesh = plsc.VectorSubcoreMesh(
        core_axis_name="c", subcore_axis_name="s",
        num_cores=NC, num_subcores=NS)

    @functools.partial(
        pl.kernel,
        out_type=jax.ShapeDtypeStruct((NW, NSEG, D), jnp.float32),
        mesh=mesh,
        scratch_types=[
            pltpu.VMEM((BLK,), jnp.int32),
            pltpu.VMEM((BLK,), jnp.int32),
            pltpu.VMEM((BLK, D), jnp.float32),
            pltpu.VMEM((BLK, D), jnp.float32),
            pltpu.VMEM((NSEG, D), jnp.float32),
            pltpu.SemaphoreType.DMA,
            pltpu.SemaphoreType.DMA,
        ],
    )
    def k(x_hbm, b_hbm, part_hbm, ids_a, ids_b, buf_a, buf_b, acc_v,
          sem_a, sem_b):
        wid = lax.axis_index("s") * NC + lax.axis_index("c")
        base = wid * CHUNK

        def blk_start(idx):
            return jnp.minimum(base + idx * BLK, LAST_START)

        neg = jnp.full((16,), NEG_INF, jnp.float32)

        def init_body(i, c):
            for f in range(NF):
                acc_v[i, pl.ds(16 * f, 16)] = neg
            return c
        lax.fori_loop(0, NSEG, init_body, 0)

        bufs = ((ids_a, buf_a, sem_a), (ids_b, buf_b, sem_b))

        def issue(idx, ids_v, buf_v, sem):
            s = blk_start(idx)
            pltpu.async_copy(b_hbm.at[pl.ds(s, BLK)], ids_v, sem)
            pltpu.async_copy(x_hbm.at[pl.ds(s, BLK)], buf_v, sem)

        def drain(idx, ids_v, buf_v, sem):
            s = blk_start(idx)
            pltpu.make_async_copy(b_hbm.at[pl.ds(s, BLK)], ids_v, sem).wait()
            pltpu.make_async_copy(x_hbm.at[pl.ds(s, BLK)], buf_v, sem).wait()

        issue(0, *bufs[0])

        def flush(seg, vacc):
            for f in range(NF):
                sl = pl.ds(16 * f, 16)
                acc_v[seg, sl] = jnp.maximum(acc_v[seg, sl], vacc[f])

        def pair_body(p, carry):
            new = carry
            for b in range(2):
                idx = 2 * p + b
                ids_v, buf_v, sem = bufs[b]

                @pl.when(idx + 1 < NBLK)
                def _():
                    issue(idx + 1, *bufs[1 - b])

                drain(idx, ids_v, buf_v, sem)

                def grp_body(t, gc, ids_v=ids_v, buf_v=buf_v):
                    cur_seg, vacc = gc
                    idv = ids_v[pl.ds(t * 16, 16)]
                    seg0 = idv[0]
                    seg15 = idv[15]

                    gmax = [buf_v[t * 16, pl.ds(16 * f, 16)]
                            for f in range(NF)]
                    for j in range(1, 16):
                        for f in range(NF):
                            gmax[f] = jnp.maximum(
                                gmax[f],
                                buf_v[t * 16 + j, pl.ds(16 * f, 16)])

                    single = seg0 == seg15
                    cont = seg0 == cur_seg

                    @pl.when(jnp.logical_and(single,
                                             jnp.logical_not(cont)))
                    def _():
                        flush(cur_seg, vacc)

                    @pl.when(jnp.logical_not(single))
                    def _():
                        flush(cur_seg, vacc)
                        for j in range(16):
                            seg = idv[j]
                            for f in range(NF):
                                sl = pl.ds(16 * f, 16)
                                acc_v[seg, sl] = jnp.maximum(
                                    acc_v[seg, sl],
                                    buf_v[t * 16 + j, sl])

                    new_vacc = tuple(
                        jnp.where(single,
                                  jnp.where(cont,
                                            jnp.maximum(vacc[f], gmax[f]),
                                            gmax[f]),
                                  neg)
                        for f in range(NF))
                    return (seg15, new_vacc)

                new = lax.fori_loop(0, BLK // 16, grp_body, new)
            return new

        init_carry = (jnp.int32(0), tuple(neg for _ in range(NF)))
        final = lax.fori_loop(0, NBLK // 2, pair_body, init_carry)

        flush(final[0], final[1])

        pltpu.sync_copy(acc_v, part_hbm.at[wid])

    return k(x, batch_i32)


def _tc_combine(part):
    def body(p_ref, o_ref):
        o_ref[...] = jnp.max(p_ref[...], axis=0)

    return pl.pallas_call(
        body,
        out_shape=jax.ShapeDtypeStruct((NSEG, D), jnp.float32),
    )(part)


def kernel(x, batch):
    part = _sc_partial_max(x, batch.astype(jnp.int32))
    return _tc_combine(part)


# speedup vs baseline: 1.7690x; 1.7690x over previous
import functools

import jax
import jax.numpy as jnp
from jax import lax
from jax.experimental import pallas as pl
from jax.experimental.pallas import tpu as pltpu
from jax.experimental.pallas import tpu_sc as plsc

N_NODES = 100000
D = 128
NF = D // 16
NSEG = 128
NC, NS = 2, 16
NW = NC * NS
CHUNK = 3128
BLK = 128
NGRP = BLK // 16
NBLK = 26
LAST_START = N_NODES - BLK
NEG_INF = float("-inf")


def _sc_partial_max(x, batch_i32):
    mesh = plsc.VectorSubcoreMesh(
        core_axis_name="c", subcore_axis_name="s",
        num_cores=NC, num_subcores=NS)

    @functools.partial(
        pl.kernel,
        out_type=jax.ShapeDtypeStruct((NW, NSEG, D), jnp.float32),
        mesh=mesh,
        scratch_types=[
            pltpu.VMEM((BLK,), jnp.int32),
            pltpu.VMEM((BLK,), jnp.int32),
            pltpu.VMEM((BLK, D), jnp.float32),
            pltpu.VMEM((BLK, D), jnp.float32),
            pltpu.VMEM((NGRP, D), jnp.float32),
            pltpu.VMEM((NSEG, D), jnp.float32),
            pltpu.SemaphoreType.DMA,
            pltpu.SemaphoreType.DMA,
        ],
    )
    def k(x_hbm, b_hbm, part_hbm, ids_a, ids_b, buf_a, buf_b, grp_v,
          acc_v, sem_a, sem_b):
        wid = lax.axis_index("s") * NC + lax.axis_index("c")
        base = wid * CHUNK

        def blk_start(idx):
            return jnp.minimum(base + idx * BLK, LAST_START)

        neg = jnp.full((16,), NEG_INF, jnp.float32)

        def init_body(i, c):
            for f in range(NF):
                acc_v[i, pl.ds(16 * f, 16)] = neg
            return c
        lax.fori_loop(0, NSEG, init_body, 0)

        bufs = ((ids_a, buf_a, sem_a), (ids_b, buf_b, sem_b))

        def issue(idx, ids_v, buf_v, sem):
            s = blk_start(idx)
            pltpu.async_copy(b_hbm.at[pl.ds(s, BLK)], ids_v, sem)
            pltpu.async_copy(x_hbm.at[pl.ds(s, BLK)], buf_v, sem)

        def drain(idx, ids_v, buf_v, sem):
            s = blk_start(idx)
            pltpu.make_async_copy(b_hbm.at[pl.ds(s, BLK)], ids_v, sem).wait()
            pltpu.make_async_copy(x_hbm.at[pl.ds(s, BLK)], buf_v, sem).wait()

        issue(0, *bufs[0])

        def pair_body(p, c):
            for b in range(2):
                idx = 2 * p + b
                ids_v, buf_v, sem = bufs[b]

                @pl.when(idx + 1 < NBLK)
                def _():
                    issue(idx + 1, *bufs[1 - b])

                drain(idx, ids_v, buf_v, sem)

                @functools.partial(plsc.parallel_loop, 0, NGRP)
                def _(t, buf_v=buf_v):
                    gmax = [buf_v[t * 16, pl.ds(16 * f, 16)]
                            for f in range(NF)]
                    for j in range(1, 16):
                        for f in range(NF):
                            gmax[f] = jnp.maximum(
                                gmax[f],
                                buf_v[t * 16 + j, pl.ds(16 * f, 16)])
                    for f in range(NF):
                        grp_v[t, pl.ds(16 * f, 16)] = gmax[f]

                def merge_body(t, mc, ids_v=ids_v, buf_v=buf_v):
                    idv = ids_v[pl.ds(t * 16, 16)]
                    seg0 = idv[0]
                    seg15 = idv[15]

                    @pl.when(seg0 == seg15)
                    def _():
                        for f in range(NF):
                            sl = pl.ds(16 * f, 16)
                            acc_v[seg15, sl] = jnp.maximum(
                                acc_v[seg15, sl], grp_v[t, sl])

                    @pl.when(seg0 != seg15)
                    def _():
                        for j in range(16):
                            seg = idv[j]
                            for f in range(NF):
                                sl = pl.ds(16 * f, 16)
                                acc_v[seg, sl] = jnp.maximum(
                                    acc_v[seg, sl],
                                    buf_v[t * 16 + j, sl])
                    return mc
                lax.fori_loop(0, NGRP, merge_body, 0)
            return c

        lax.fori_loop(0, NBLK // 2, pair_body, 0)

        pltpu.sync_copy(acc_v, part_hbm.at[wid])

    return k(x, batch_i32)


def _tc_combine(part):
    def body(p_ref, o_ref):
        o_ref[...] = jnp.max(p_ref[...], axis=0)

    return pl.pallas_call(
        body,
        out_shape=jax.ShapeDtypeStruct((NSEG, D), jnp.float32),
    )(part)


def kernel(x, batch):
    part = _sc_partial_max(x, batch.astype(jnp.int32))
    return _tc_combine(part)
